# baseline (speedup 1.0000x reference)
import functools

import jax
import jax.numpy as jnp
from jax import lax
from jax.experimental import pallas as pl
from jax.experimental.pallas import tpu as pltpu
from jax.experimental.pallas import tpu_sc as plsc

N_PERIODS = 10
EMB = 64
LANES = 16
NC, NS = 2, 16
NW = NC * NS
CHUNK = 128


def _tmf_body(user_hbm, item_hbm, age_hbm, utab, itab, dtab, out_hbm,
              uidx, iidx, aidx, didx, urows, irows, drows, outv,
              sem0, sem1, sem2, sem3):
    n_chunks = uidx.shape[0]
    per_w = n_chunks * CHUNK
    wid = lax.axis_index("s") * NC + lax.axis_index("c")
    row0 = wid * n_chunks

    pltpu.sync_copy(user_hbm.at[pl.ds(row0, n_chunks)], uidx)
    pltpu.sync_copy(item_hbm.at[pl.ds(row0, n_chunks)], iidx)
    pltpu.sync_copy(age_hbm.at[pl.ds(row0, n_chunks)], aidx)

    for j in range(n_chunks):
        for c in range(CHUNK // LANES):
            sl = pl.ds(c * LANES, LANES)
            didx[j, sl] = iidx[j, sl] * N_PERIODS + aidx[j, sl]

    sems = (sem0, sem1, sem2, sem3)
    copies = []
    for j in range(n_chunks):
        copies.append((
            pltpu.async_copy(utab.at[uidx.at[j]], urows.at[j], sems[j]),
            pltpu.async_copy(itab.at[iidx.at[j]], irows.at[j], sems[j]),
            pltpu.async_copy(dtab.at[didx.at[j]], drows.at[j], sems[j]),
        ))

    lane = lax.iota(jnp.int32, LANES)
    for j in range(n_chunks):
        for cpy in copies[j]:
            cpy.wait()
        uref, iref, dref = urows.at[j], irows.at[j], drows.at[j]

        @pl.loop(0, CHUNK // LANES)
        def _group(g, _j=j, _u=uref, _i=iref, _d=dref):
            rvec = g * LANES + lane
            acc = [jnp.zeros((LANES,), jnp.float32) for _ in range(4)]
            for d in range(EMB):
                dvec = jnp.full((LANES,), d, jnp.int32)
                u = plsc.load_gather(_u, [rvec, dvec])
                v = (plsc.load_gather(_i, [rvec, dvec])
                     + plsc.load_gather(_d, [rvec, dvec]))
                acc[d % 4] = acc[d % 4] + u * v
            outv[pl.ds(_j * CHUNK + g * LANES, LANES)] = (
                (acc[0] + acc[1]) + (acc[2] + acc[3]))

    pltpu.sync_copy(outv, out_hbm.at[pl.ds(wid * per_w, per_w)])


def kernel(user, item, itemage, user_table, item_table, item_dyn_table):
    batch = user.shape[0]
    n_rows = batch // CHUNK
    n_chunks = n_rows // NW
    per_w = n_chunks * CHUNK

    user2 = user.astype(jnp.int32).reshape(n_rows, CHUNK)
    item2 = item.astype(jnp.int32).reshape(n_rows, CHUNK)
    age2 = itemage.astype(jnp.int32).reshape(n_rows, CHUNK)

    run = functools.partial(
        pl.kernel,
        out_type=jax.ShapeDtypeStruct((batch,), jnp.float32),
        mesh=plsc.VectorSubcoreMesh(
            core_axis_name="c", subcore_axis_name="s",
            num_cores=NC, num_subcores=NS),
        compiler_params=pltpu.CompilerParams(
            needs_layout_passes=False, use_tc_tiling_on_sc=False),
        scratch_types=[
            pltpu.VMEM((n_chunks, CHUNK), jnp.int32),
            pltpu.VMEM((n_chunks, CHUNK), jnp.int32),
            pltpu.VMEM((n_chunks, CHUNK), jnp.int32),
            pltpu.VMEM((n_chunks, CHUNK), jnp.int32),
            pltpu.VMEM((n_chunks, CHUNK, EMB), jnp.float32),
            pltpu.VMEM((n_chunks, CHUNK, EMB), jnp.float32),
            pltpu.VMEM((n_chunks, CHUNK, EMB), jnp.float32),
            pltpu.VMEM((per_w,), jnp.float32),
            pltpu.SemaphoreType.DMA,
            pltpu.SemaphoreType.DMA,
            pltpu.SemaphoreType.DMA,
            pltpu.SemaphoreType.DMA,
        ],
    )(_tmf_body)
    return run(user2, item2, age2, user_table, item_table, item_dyn_table)


# speedup vs baseline: 1.0114x; 1.0114x over previous
import functools

import jax
import jax.numpy as jnp
from jax import lax
from jax.experimental import pallas as pl
from jax.experimental.pallas import tpu as pltpu
from jax.experimental.pallas import tpu_sc as plsc

N_PERIODS = 10
EMB = 64
L = 16
NC, NS = 2, 16
NW = NC * NS
CHUNK = 128


def _tmf_body(user_hbm, item_hbm, age_hbm, ut2, it2, dt2, out_hbm,
              uidx, iidx, aidx, uh, ih, dh, outv,
              bu0, bi0, bd0, bu1, bi1, bd1, sem0, sem1):
    n_chunks = uidx.shape[0]
    per_w = n_chunks * CHUNK
    wid = lax.axis_index("s") * NC + lax.axis_index("c")
    row0 = wid * n_chunks

    pltpu.sync_copy(user_hbm.at[pl.ds(row0, n_chunks)], uidx)
    pltpu.sync_copy(item_hbm.at[pl.ds(row0, n_chunks)], iidx)
    pltpu.sync_copy(age_hbm.at[pl.ds(row0, n_chunks)], aidx)

    for j in range(n_chunks):
        for c in range(CHUNK // L):
            sl = pl.ds(c * L, L)
            u = uidx[j, sl]
            i = iidx[j, sl]
            dyn = i * N_PERIODS + aidx[j, sl]
            uh[j, sl] = lax.shift_right_logical(u, 1)
            ih[j, sl] = lax.shift_right_logical(i, 1)
            dh[j, sl] = lax.shift_right_logical(dyn, 1)

    bufs = ((bu0, bi0, bd0), (bu1, bi1, bd1))
    sems = (sem0, sem1)

    def fire(j):
        k = j % 2
        return (
            pltpu.async_copy(ut2.at[uh.at[j]], bufs[k][0], sems[k]),
            pltpu.async_copy(it2.at[ih.at[j]], bufs[k][1], sems[k]),
            pltpu.async_copy(dt2.at[dh.at[j]], bufs[k][2], sems[k]),
        )

    lane = lax.iota(jnp.int32, L)
    one = jnp.full((L,), 1, jnp.int32)
    pend = fire(0)
    for j in range(n_chunks):
        nxt = fire(j + 1) if j + 1 < n_chunks else None
        for cp in pend:
            cp.wait()
        bu, bi, bd = bufs[j % 2]

        @pl.loop(0, CHUNK // L)
        def _group(g, _j=j, _bu=bu, _bi=bi, _bd=bd):
            sl = pl.ds(g * L, L)
            u = uidx[_j, sl]
            i = iidx[_j, sl]
            dyn = i * N_PERIODS + aidx[_j, sl]
            ub = (u & one) * EMB
            ib = (i & one) * EMB
            db = (dyn & one) * EMB
            rvec = g * L + lane
            acc = [jnp.zeros((L,), jnp.float32) for _ in range(4)]
            for d in range(EMB):
                uv = plsc.load_gather(_bu, [rvec, ub + d])
                iv = plsc.load_gather(_bi, [rvec, ib + d])
                dv = plsc.load_gather(_bd, [rvec, db + d])
                acc[d % 4] = acc[d % 4] + uv * (iv + dv)
            outv[pl.ds(_j * CHUNK + g * L, L)] = (
                (acc[0] + acc[1]) + (acc[2] + acc[3]))
        pend = nxt

    pltpu.sync_copy(outv, out_hbm.at[pl.ds(wid * per_w, per_w)])


def kernel(user, item, itemage, user_table, item_table, item_dyn_table):
    batch = user.shape[0]
    n_rows = batch // CHUNK
    n_chunks = n_rows // NW

    user2 = user.astype(jnp.int32).reshape(n_rows, CHUNK)
    item2 = item.astype(jnp.int32).reshape(n_rows, CHUNK)
    age2 = itemage.astype(jnp.int32).reshape(n_rows, CHUNK)
    ut2 = user_table.reshape(-1, 2 * EMB)
    it2 = item_table.reshape(-1, 2 * EMB)
    dt2 = item_dyn_table.reshape(-1, 2 * EMB)

    run = functools.partial(
        pl.kernel,
        out_type=jax.ShapeDtypeStruct((batch,), jnp.float32),
        mesh=plsc.VectorSubcoreMesh(
            core_axis_name="c", subcore_axis_name="s",
            num_cores=NC, num_subcores=NS),
        compiler_params=pltpu.CompilerParams(
            needs_layout_passes=False, use_tc_tiling_on_sc=True),
        scratch_types=[
            pltpu.VMEM((n_chunks, CHUNK), jnp.int32),
            pltpu.VMEM((n_chunks, CHUNK), jnp.int32),
            pltpu.VMEM((n_chunks, CHUNK), jnp.int32),
            pltpu.VMEM((n_chunks, CHUNK), jnp.int32),
            pltpu.VMEM((n_chunks, CHUNK), jnp.int32),
            pltpu.VMEM((n_chunks, CHUNK), jnp.int32),
            pltpu.VMEM((n_chunks * CHUNK,), jnp.float32),
            pltpu.VMEM((CHUNK, 2 * EMB), jnp.float32),
            pltpu.VMEM((CHUNK, 2 * EMB), jnp.float32),
            pltpu.VMEM((CHUNK, 2 * EMB), jnp.float32),
            pltpu.VMEM((CHUNK, 2 * EMB), jnp.float32),
            pltpu.VMEM((CHUNK, 2 * EMB), jnp.float32),
            pltpu.VMEM((CHUNK, 2 * EMB), jnp.float32),
            pltpu.SemaphoreType.DMA,
            pltpu.SemaphoreType.DMA,
        ],
    )(_tmf_body)
    return run(user2, item2, age2, ut2, it2, dt2)


# speedup vs baseline: 1.9611x; 1.9390x over previous
import functools

import jax
import jax.numpy as jnp
from jax import lax
from jax.experimental import pallas as pl
from jax.experimental.pallas import tpu as pltpu
from jax.experimental.pallas import tpu_sc as plsc

N_PERIODS = 10
EMB = 64
L = 16
NC, NS = 2, 16
NW = NC * NS
B = 16384

N_USERS = 1000000
N_ITEMS = 100000
N_DYN = N_ITEMS * N_PERIODS

U_COLS = (N_USERS // 128)
I_COLS = (N_ITEMS // 128)
U_WORKERS, D_WORKERS, I_WORKERS = 14, 14, 4
U_PER_W = U_COLS // U_WORKERS
I_PER_W = -(-I_COLS // I_WORKERS)
U_CLAMP = U_COLS * 128 - 1
I_CLAMP = I_COLS * 128 - 1
U_TAIL, I_TAIL = U_COLS * 128, I_COLS * 128

HIT_CAP = 6144
GBUF_CAP = 10240
SLOT_CAP = 64
SENT = 0x7FFFFFF0


def _pc(mask):
    c = plsc.all_reduce_population_count(mask)
    return c[0] if getattr(c, "ndim", 0) else c


def _compact_step(count, vals, bvals, mask, out_v, out_b, base):
    inc = plsc.cumsum(mask.astype(jnp.int32))
    pos = base + count + inc - 1
    plsc.store_scatter(out_v, [pos], vals, mask=mask)
    plsc.store_scatter(out_b, [pos], bvals, mask=mask)
    return count + _pc(mask)


def _p1_body(user_hbm, item_hbm, age_hbm, utT, itT, dtT,
             scr_u, scr_i, scr_d,
             ids, tmp, hit_v, hit_b, grp_v, grp_b,
             sb0, sb1, sb2, sb3, stg0, stg1, jb0, jb1, bb0, bb1,
             m2s, ss0, ss1, ss2, ss3, sc0, sc1):
    wid = lax.axis_index("s") * NC + lax.axis_index("c")
    lane = lax.iota(jnp.int32, L)

    is_u = wid < U_WORKERS
    is_d = (wid >= U_WORKERS) & (wid < U_WORKERS + D_WORKERS)
    is_i = wid >= U_WORKERS + D_WORKERS

    @pl.when(is_u)
    def _():
        pltpu.sync_copy(user_hbm, ids)

        @pl.loop(0, B // L)
        def _(i):
            sl = pl.ds(i * L, L)
            ids[sl] = jnp.minimum(ids[sl], U_CLAMP)

    @pl.when(is_d)
    def _():
        pltpu.sync_copy(item_hbm, ids)
        pltpu.sync_copy(age_hbm, tmp)

        @pl.loop(0, B // L)
        def _(i):
            sl = pl.ds(i * L, L)
            ids[sl] = jnp.minimum(ids[sl] * N_PERIODS + tmp[sl], U_CLAMP)

    @pl.when(is_i)
    def _():
        pltpu.sync_copy(item_hbm, ids)

        @pl.loop(0, B // L)
        def _(i):
            sl = pl.ds(i * L, L)
            ids[sl] = jnp.minimum(ids[sl], I_CLAMP)

    sub = jnp.where(is_u, wid,
                    jnp.where(is_d, wid - U_WORKERS,
                              wid - U_WORKERS - D_WORKERS))
    per_w = jnp.where(is_i, I_PER_W, U_PER_W)
    lo = sub * per_w * 128
    nch = jnp.where(is_i, jnp.minimum(I_PER_W, I_COLS - sub * I_PER_W),
                    U_PER_W)
    gstride = jnp.where(is_i, 768, 256)
    hi = lo + nch * 128

    sent_v = jnp.full((L,), SENT, jnp.int32)

    @pl.loop(0, HIT_CAP // L)
    def _(i):
        hit_v[pl.ds(i * L, L)] = sent_v

    @pl.loop(0, GBUF_CAP // L)
    def _(i):
        grp_v[pl.ds(i * L, L)] = sent_v

    @pl.loop(0, B // L, init_carry=jnp.int32(0))
    def m1(i, cnt):
        sl = pl.ds(i * L, L)
        v = ids[sl]
        mask = (v >= lo) & (v < hi)
        return _compact_step(cnt, v, i * L + lane, mask, hit_v, hit_b, 0)

    m1t = (m1 + L - 1) // L

    n_l1 = (nch + 15) >> 4

    @pl.loop(0, n_l1)
    def _(g):
        glo = lo + g * (16 * 128)
        ghi = glo + 16 * 128

        @pl.loop(0, m1t, init_carry=jnp.int32(0))
        def m2(t, cnt):
            sl = pl.ds(t * L, L)
            v = hit_v[sl]
            mask = (v >= glo) & (v < ghi)
            return _compact_step(cnt, v, hit_b[sl], mask,
                                 grp_v, grp_b, g * gstride)

        m2s[g] = m2

    sbufs = (sb0, sb1, sb2, sb3)
    ssems = (ss0, ss1, ss2, ss3)
    stgs = (stg0, stg1)
    jbs = (jb0, jb1)
    bbs = (bb0, bb1)
    csems = (sc0, sc1)

    def tab_of(fn):
        @pl.when(is_u)
        def _():
            fn(utT, scr_u)

        @pl.when(is_d)
        def _():
            fn(dtT, scr_d)

        @pl.when(is_i)
        def _():
            fn(itT, scr_i)

    def fire_stream(tab, c, k):
        off = pl.multiple_of(lo + c * 128, 128)
        pltpu.async_copy(tab.at[:, pl.ds(off, 128)], sbufs[k], ssems[k])

    def drain_stream(tab, k):
        pltpu.make_async_copy(tab.at[:, pl.ds(0, 128)], sbufs[k],
                              ssems[k]).wait()

    def fire_scatter(scr, k):
        pltpu.async_copy(stgs[k], scr.at[plsc.Indices(bbs[k],
                                                      ignored_value=-1)],
                         csems[k])

    def wait_scatter(scr, k):
        pltpu.make_async_copy(stgs[k], scr.at[plsc.Indices(bbs[k],
                                                           ignored_value=-1)],
                              csems[k]).wait()

    def prologue(tab, scr):
        for k in range(4):
            fire_stream(tab, jnp.int32(k), k)

    tab_of(prologue)

    neg1 = jnp.full((L,), -1, jnp.int32)
    zero = jnp.zeros((L,), jnp.int32)

    def do_chunk(tab, scr, c, k):
        sk = k % 2

        @pl.when(c >= 2)
        def _():
            wait_scatter(scr, sk)

        for t in range(SLOT_CAP // L):
            jbs[sk][pl.ds(t * L, L)] = zero
            bbs[sk][pl.ds(t * L, L)] = neg1

        g = c >> 4
        base = g * gstride
        m2 = m2s[g]
        m2t = (m2 + L - 1) // L
        clo = lo + c * 128

        @pl.loop(0, m2t, init_carry=jnp.int32(0))
        def m3(t, cnt):
            sl = pl.ds(base + t * L, L)
            v = grp_v[sl]
            mask = (v >= clo) & (v < clo + 128)
            return _compact_step(cnt, v - clo, grp_b[sl], mask,
                                 jbs[sk], bbs[sk], 0)

        drain_stream(tab, k)

        m3t = jnp.minimum((m3 + L - 1) // L, SLOT_CAP // L)

        @pl.loop(0, m3t)
        def _(t):
            jv = jbs[sk][pl.ds(t * L, L)]
            for j in range(L):
                col = jnp.full((L,), jv[j], jnp.int32)
                slot = t * L + j
                for cch in range(EMB // L):
                    dvec = cch * L + lane
                    stgs[sk][slot, pl.ds(cch * L, L)] = (
                        plsc.load_gather(sbufs[k], [dvec, col]))

        fire_scatter(scr, sk)

    def stream_loop(tab, scr):
        @pl.loop(0, (U_PER_W + 3) // 4 * 4, step=4)
        def _(c0):
            for par in range(4):
                c = c0 + par

                @pl.when(c < nch)
                def _(c=c, par=par):
                    do_chunk(tab, scr, c, par)

                @pl.when(c + 4 < nch)
                def _(c=c, par=par):
                    fire_stream(tab, c + 4, par)

        wait_scatter(scr, 0)
        wait_scatter(scr, 1)

    tab_of(stream_loop)


def _p2_body(user_hbm, item_hbm, age_hbm, scr_u, scr_i, scr_d,
             tail_u, tail_i, tail_d, out_hbm,
             uv, iv, av, tu, ti, td, bu, bi, bd, outv):
    wid = lax.axis_index("s") * NC + lax.axis_index("c")
    lane = lax.iota(jnp.int32, L)
    per_w = B // NW
    base = wid * per_w

    pltpu.sync_copy(user_hbm.at[pl.ds(base, per_w)], uv)
    pltpu.sync_copy(item_hbm.at[pl.ds(base, per_w)], iv)
    pltpu.sync_copy(age_hbm.at[pl.ds(base, per_w)], av)
    pltpu.sync_copy(tail_u, tu)
    pltpu.sync_copy(tail_i, ti)
    pltpu.sync_copy(tail_d, td)

    half = per_w // 2
    for h in range(2):
        pltpu.sync_copy(scr_u.at[pl.ds(base + h * half, half)], bu)
        pltpu.sync_copy(scr_i.at[pl.ds(base + h * half, half)], bi)
        pltpu.sync_copy(scr_d.at[pl.ds(base + h * half, half)], bd)

        @pl.loop(0, half // L)
        def _(g, _h=h):
            sl = pl.ds(_h * half + g * L, L)
            u = uv[sl]
            it = iv[sl]
            dyn = it * N_PERIODS + av[sl]
            um = u >= U_TAIL
            im = it >= I_TAIL
            dm = dyn >= U_TAIL
            ur = jnp.clip(u - U_TAIL, 0, N_USERS - U_TAIL - 1)
            ir = jnp.clip(it - I_TAIL, 0, N_ITEMS - I_TAIL - 1)
            dr = jnp.clip(dyn - U_TAIL, 0, N_DYN - U_TAIL - 1)
            rvec = g * L + lane
            acc = [jnp.zeros((L,), jnp.float32) for _ in range(4)]
            for d in range(EMB):
                dvec = jnp.full((L,), d, jnp.int32)
                uu = jnp.where(um, plsc.load_gather(tu, [ur, dvec]),
                               plsc.load_gather(bu, [rvec, dvec]))
                ii = jnp.where(im, plsc.load_gather(ti, [ir, dvec]),
                               plsc.load_gather(bi, [rvec, dvec]))
                dd = jnp.where(dm, plsc.load_gather(td, [dr, dvec]),
                               plsc.load_gather(bd, [rvec, dvec]))
                acc[d % 4] = acc[d % 4] + uu * (ii + dd)
            outv[pl.ds(_h * half + g * L, L)] = (
                (acc[0] + acc[1]) + (acc[2] + acc[3]))

    pltpu.sync_copy(outv, out_hbm.at[pl.ds(base, per_w)])


def kernel(user, item, itemage, user_table, item_table, item_dyn_table):
    user1 = user.astype(jnp.int32)
    item1 = item.astype(jnp.int32)
    age1 = itemage.astype(jnp.int32)
    mesh = plsc.VectorSubcoreMesh(core_axis_name="c", subcore_axis_name="s",
                                  num_cores=NC, num_subcores=NS)
    params = pltpu.CompilerParams(needs_layout_passes=False,
                                  use_tc_tiling_on_sc=True)

    p1 = functools.partial(
        pl.kernel,
        out_type=(jax.ShapeDtypeStruct((B, 128), jnp.float32),
                  jax.ShapeDtypeStruct((B, 128), jnp.float32),
                  jax.ShapeDtypeStruct((B, 128), jnp.float32)),
        mesh=mesh,
        compiler_params=params,
        scratch_types=[
            pltpu.VMEM((B,), jnp.int32),
            pltpu.VMEM((B,), jnp.int32),
            pltpu.VMEM((HIT_CAP,), jnp.int32),
            pltpu.VMEM((HIT_CAP,), jnp.int32),
            pltpu.VMEM((GBUF_CAP,), jnp.int32),
            pltpu.VMEM((GBUF_CAP,), jnp.int32),
            pltpu.VMEM((EMB, 128), jnp.float32),
            pltpu.VMEM((EMB, 128), jnp.float32),
            pltpu.VMEM((EMB, 128), jnp.float32),
            pltpu.VMEM((EMB, 128), jnp.float32),
            pltpu.VMEM((SLOT_CAP, 128), jnp.float32),
            pltpu.VMEM((SLOT_CAP, 128), jnp.float32),
            pltpu.VMEM((SLOT_CAP,), jnp.int32),
            pltpu.VMEM((SLOT_CAP,), jnp.int32),
            pltpu.VMEM((SLOT_CAP,), jnp.int32),
            pltpu.VMEM((SLOT_CAP,), jnp.int32),
            pltpu.SMEM((64,), jnp.int32),
            pltpu.SemaphoreType.DMA,
            pltpu.SemaphoreType.DMA,
            pltpu.SemaphoreType.DMA,
            pltpu.SemaphoreType.DMA,
            pltpu.SemaphoreType.DMA,
            pltpu.SemaphoreType.DMA,
        ],
    )(_p1_body)
    scr_u, scr_i, scr_d = p1(user1, item1, age1,
                             user_table.T, item_table.T, item_dyn_table.T)

    tail_u = user_table[U_TAIL:]
    tail_i = item_table[I_TAIL:]
    tail_d = item_dyn_table[U_TAIL:]

    per_w = B // NW
    p2 = functools.partial(
        pl.kernel,
        out_type=jax.ShapeDtypeStruct((B,), jnp.float32),
        mesh=mesh,
        compiler_params=params,
        scratch_types=[
            pltpu.VMEM((per_w,), jnp.int32),
            pltpu.VMEM((per_w,), jnp.int32),
            pltpu.VMEM((per_w,), jnp.int32),
            pltpu.VMEM((N_USERS - U_TAIL, EMB), jnp.float32),
            pltpu.VMEM((N_ITEMS - I_TAIL, EMB), jnp.float32),
            pltpu.VMEM((N_DYN - U_TAIL, EMB), jnp.float32),
            pltpu.VMEM((per_w // 2, 128), jnp.float32),
            pltpu.VMEM((per_w // 2, 128), jnp.float32),
            pltpu.VMEM((per_w // 2, 128), jnp.float32),
            pltpu.VMEM((per_w,), jnp.float32),
        ],
    )(_p2_body)
    return p2(user1, item1, age1, scr_u, scr_i, scr_d,
              tail_u, tail_i, tail_d)


# speedup vs baseline: 2.5870x; 1.3191x over previous
import functools

import jax
import jax.numpy as jnp
from jax import lax
from jax.experimental import pallas as pl
from jax.experimental.pallas import tpu as pltpu
from jax.experimental.pallas import tpu_sc as plsc

N_PERIODS = 10
EMB = 64
L = 16
NC, NS = 2, 16
NW = NC * NS
B = 16384

N_USERS = 1000000
N_ITEMS = 100000
N_DYN = N_ITEMS * N_PERIODS

U_COLS = (N_USERS // 128)
I_COLS = (N_ITEMS // 128)
U_WORKERS, D_WORKERS, I_WORKERS = 14, 14, 4
CW = 256
U_CH_W = U_COLS // U_WORKERS // 2
I_CHUNKS = -(-I_COLS // 2)
I_CH_W = -(-I_CHUNKS // I_WORKERS)
U_CLAMP = U_COLS * 128 - 1
I_CLAMP = I_COLS * 128 - 1
U_TAIL, I_TAIL = U_COLS * 128, I_COLS * 128

HIT_CAP = 6144
GBUF_CAP = 10240
SLOT_CAP = 96
SENT = 0x7FFFFFF0


def _pc(mask):
    c = plsc.all_reduce_population_count(mask)
    return c[0] if getattr(c, "ndim", 0) else c


def _compact_step(count, vals, bvals, mask, out_v, out_b, base):
    inc = plsc.cumsum(mask.astype(jnp.int32))
    pos = base + count + inc - 1
    plsc.store_scatter(out_v, [pos], vals, mask=mask)
    plsc.store_scatter(out_b, [pos], bvals, mask=mask)
    return count + _pc(mask)


def _p1_body(user_hbm, item_hbm, age_hbm, utT, itT, dtT,
             scr_u, scr_i, scr_d,
             ids, tmp, hit_v, hit_b, grp_v, grp_b,
             sb0, sb1, sb2, stg0, stg1, jb0, jb1, bb0, bb1,
             m2s, ss0, ss1, ss2, sc0, sc1):
    wid = lax.axis_index("s") * NC + lax.axis_index("c")
    lane = lax.iota(jnp.int32, L)

    is_u = wid < U_WORKERS
    is_d = (wid >= U_WORKERS) & (wid < U_WORKERS + D_WORKERS)
    is_i = wid >= U_WORKERS + D_WORKERS

    @pl.when(is_u)
    def _():
        pltpu.sync_copy(user_hbm, ids)

        @pl.loop(0, B // L)
        def _(i):
            sl = pl.ds(i * L, L)
            ids[sl] = jnp.minimum(ids[sl], U_CLAMP)

    @pl.when(is_d)
    def _():
        pltpu.sync_copy(item_hbm, ids)
        for st in range(4):
            pltpu.sync_copy(age_hbm.at[pl.ds(st * 4096, 4096)], tmp)

            @pl.loop(0, 4096 // L)
            def _(i, _st=st):
                sl = pl.ds(_st * 4096 + i * L, L)
                slt = pl.ds(i * L, L)
                ids[sl] = jnp.minimum(ids[sl] * N_PERIODS + tmp[slt], U_CLAMP)

    @pl.when(is_i)
    def _():
        pltpu.sync_copy(item_hbm, ids)

        @pl.loop(0, B // L)
        def _(i):
            sl = pl.ds(i * L, L)
            ids[sl] = jnp.minimum(ids[sl], I_CLAMP)

    sub = jnp.where(is_u, wid,
                    jnp.where(is_d, wid - U_WORKERS,
                              wid - U_WORKERS - D_WORKERS))
    per_w = jnp.where(is_i, I_CH_W, U_CH_W)
    lo = sub * per_w * CW
    nch = jnp.where(is_i, jnp.minimum(I_CH_W, I_CHUNKS - sub * I_CH_W),
                    U_CH_W)
    gstride = jnp.where(is_i, 768, 256)
    hi = jnp.minimum(lo + nch * CW, jnp.where(is_i, I_TAIL, U_TAIL))

    sent_v = jnp.full((L,), SENT, jnp.int32)

    @pl.loop(0, HIT_CAP // L)
    def _(i):
        hit_v[pl.ds(i * L, L)] = sent_v

    @pl.loop(0, GBUF_CAP // L)
    def _(i):
        grp_v[pl.ds(i * L, L)] = sent_v

    @pl.loop(0, B // L, init_carry=jnp.int32(0))
    def m1(i, cnt):
        sl = pl.ds(i * L, L)
        v = ids[sl]
        mask = (v >= lo) & (v < hi)
        return _compact_step(cnt, v, i * L + lane, mask, hit_v, hit_b, 0)

    m1t = (m1 + L - 1) // L

    n_l1 = (nch + 7) >> 3

    @pl.loop(0, n_l1)
    def _(g):
        glo = lo + g * (8 * CW)
        ghi = glo + 8 * CW

        @pl.loop(0, m1t, init_carry=jnp.int32(0))
        def m2(t, cnt):
            sl = pl.ds(t * L, L)
            v = hit_v[sl]
            mask = (v >= glo) & (v < ghi)
            return _compact_step(cnt, v, hit_b[sl], mask,
                                 grp_v, grp_b, g * gstride)

        m2s[g] = m2

    sbufs = (sb0, sb1, sb2)
    ssems = (ss0, ss1, ss2)
    stgs = (stg0, stg1)
    jbs = (jb0, jb1)
    bbs = (bb0, bb1)
    csems = (sc0, sc1)

    def tab_of(fn):
        @pl.when(is_u)
        def _():
            fn(utT, scr_u)

        @pl.when(is_d)
        def _():
            fn(dtT, scr_d)

        @pl.when(is_i)
        def _():
            fn(itT, scr_i)

    def fire_stream(tab, c, k):
        off = pl.multiple_of(lo + c * CW, 128)
        pltpu.async_copy(tab.at[:, pl.ds(off, CW)], sbufs[k], ssems[k])

    def drain_stream(tab, k):
        pltpu.make_async_copy(tab.at[:, pl.ds(0, CW)], sbufs[k],
                              ssems[k]).wait()

    def fire_scatter(scr, k):
        pltpu.async_copy(stgs[k], scr.at[plsc.Indices(bbs[k],
                                                      ignored_value=-1)],
                         csems[k])

    def wait_scatter(scr, k):
        pltpu.make_async_copy(stgs[k], scr.at[plsc.Indices(bbs[k],
                                                           ignored_value=-1)],
                              csems[k]).wait()

    def prologue(tab, scr):
        for k in range(3):
            fire_stream(tab, jnp.int32(k), k)

    tab_of(prologue)

    neg1 = jnp.full((L,), -1, jnp.int32)
    zero = jnp.zeros((L,), jnp.int32)

    def do_chunk(tab, scr, c, k):
        sk = k % 2

        @pl.when(c >= 2)
        def _():
            wait_scatter(scr, sk)

        for t in range(SLOT_CAP // L):
            jbs[sk][pl.ds(t * L, L)] = zero
            bbs[sk][pl.ds(t * L, L)] = neg1

        g = c >> 3
        base = g * gstride
        m2 = m2s[g]
        m2t = (m2 + L - 1) // L
        clo = lo + c * CW

        @pl.loop(0, m2t, init_carry=jnp.int32(0))
        def m3(t, cnt):
            sl = pl.ds(base + t * L, L)
            v = grp_v[sl]
            mask = (v >= clo) & (v < clo + CW)
            return _compact_step(cnt, v - clo, grp_b[sl], mask,
                                 jbs[sk], bbs[sk], 0)

        drain_stream(tab, k)

        m3t = jnp.minimum((m3 + L - 1) // L, SLOT_CAP // L)

        @pl.loop(0, m3t)
        def _(t):
            jv = jbs[sk][pl.ds(t * L, L)]
            for j in range(L):
                col = jnp.full((L,), jv[j], jnp.int32)
                slot = t * L + j
                for cch in range(EMB // L):
                    dvec = cch * L + lane
                    stgs[sk][slot, pl.ds(cch * L, L)] = (
                        plsc.load_gather(sbufs[k], [dvec, col]))

        fire_scatter(scr, sk)

    def stream_loop(tab, scr):
        @pl.loop(0, U_CH_W, step=3)
        def _(c0):
            for par in range(3):
                c = c0 + par

                @pl.when(c < nch)
                def _(c=c, par=par):
                    do_chunk(tab, scr, c, par)

                @pl.when(c + 3 < nch)
                def _(c=c, par=par):
                    fire_stream(tab, c + 3, par)

        wait_scatter(scr, 0)
        wait_scatter(scr, 1)

    tab_of(stream_loop)


def _p2_body(user_hbm, item_hbm, age_hbm, scr_u, scr_i, scr_d,
             tail_u, tail_i, tail_d, out_hbm,
             uv, iv, av, tu, ti, td, bu, bi, bd, outv):
    wid = lax.axis_index("s") * NC + lax.axis_index("c")
    lane = lax.iota(jnp.int32, L)
    per_w = B // NW
    base = wid * per_w

    pltpu.sync_copy(user_hbm.at[pl.ds(base, per_w)], uv)
    pltpu.sync_copy(item_hbm.at[pl.ds(base, per_w)], iv)
    pltpu.sync_copy(age_hbm.at[pl.ds(base, per_w)], av)
    pltpu.sync_copy(tail_u, tu)
    pltpu.sync_copy(tail_i, ti)
    pltpu.sync_copy(tail_d, td)

    half = per_w // 2
    for h in range(2):
        pltpu.sync_copy(scr_u.at[pl.ds(base + h * half, half)], bu)
        pltpu.sync_copy(scr_i.at[pl.ds(base + h * half, half)], bi)
        pltpu.sync_copy(scr_d.at[pl.ds(base + h * half, half)], bd)

        @pl.loop(0, half // L)
        def _(g, _h=h):
            sl = pl.ds(_h * half + g * L, L)
            u = uv[sl]
            it = iv[sl]
            dyn = it * N_PERIODS + av[sl]
            um = u >= U_TAIL
            im = it >= I_TAIL
            dm = dyn >= U_TAIL
            ur = jnp.clip(u - U_TAIL, 0, N_USERS - U_TAIL - 1)
            ir = jnp.clip(it - I_TAIL, 0, N_ITEMS - I_TAIL - 1)
            dr = jnp.clip(dyn - U_TAIL, 0, N_DYN - U_TAIL - 1)
            rvec = g * L + lane
            acc = [jnp.zeros((L,), jnp.float32) for _ in range(4)]
            for d in range(EMB):
                dvec = jnp.full((L,), d, jnp.int32)
                uu = jnp.where(um, plsc.load_gather(tu, [ur, dvec]),
                               plsc.load_gather(bu, [rvec, dvec]))
                ii = jnp.where(im, plsc.load_gather(ti, [ir, dvec]),
                               plsc.load_gather(bi, [rvec, dvec]))
                dd = jnp.where(dm, plsc.load_gather(td, [dr, dvec]),
                               plsc.load_gather(bd, [rvec, dvec]))
                acc[d % 4] = acc[d % 4] + uu * (ii + dd)
            outv[pl.ds(_h * half + g * L, L)] = (
                (acc[0] + acc[1]) + (acc[2] + acc[3]))

    pltpu.sync_copy(outv, out_hbm.at[pl.ds(base, per_w)])


def kernel(user, item, itemage, user_table, item_table, item_dyn_table):
    user1 = user.astype(jnp.int32)
    item1 = item.astype(jnp.int32)
    age1 = itemage.astype(jnp.int32)
    mesh = plsc.VectorSubcoreMesh(core_axis_name="c", subcore_axis_name="s",
                                  num_cores=NC, num_subcores=NS)
    params = pltpu.CompilerParams(needs_layout_passes=False,
                                  use_tc_tiling_on_sc=True)

    p1 = functools.partial(
        pl.kernel,
        out_type=(jax.ShapeDtypeStruct((B, 128), jnp.float32),
                  jax.ShapeDtypeStruct((B, 128), jnp.float32),
                  jax.ShapeDtypeStruct((B, 128), jnp.float32)),
        mesh=mesh,
        compiler_params=params,
        scratch_types=[
            pltpu.VMEM((B,), jnp.int32),
            pltpu.VMEM((4096,), jnp.int32),
            pltpu.VMEM((HIT_CAP,), jnp.int32),
            pltpu.VMEM((HIT_CAP,), jnp.int32),
            pltpu.VMEM((GBUF_CAP,), jnp.int32),
            pltpu.VMEM((GBUF_CAP,), jnp.int32),
            pltpu.VMEM((EMB, CW), jnp.float32),
            pltpu.VMEM((EMB, CW), jnp.float32),
            pltpu.VMEM((EMB, CW), jnp.float32),
            pltpu.VMEM((SLOT_CAP, 128), jnp.float32),
            pltpu.VMEM((SLOT_CAP, 128), jnp.float32),
            pltpu.VMEM((SLOT_CAP,), jnp.int32),
            pltpu.VMEM((SLOT_CAP,), jnp.int32),
            pltpu.VMEM((SLOT_CAP,), jnp.int32),
            pltpu.VMEM((SLOT_CAP,), jnp.int32),
            pltpu.SMEM((64,), jnp.int32),
            pltpu.SemaphoreType.DMA,
            pltpu.SemaphoreType.DMA,
            pltpu.SemaphoreType.DMA,
            pltpu.SemaphoreType.DMA,
            pltpu.SemaphoreType.DMA,
        ],
    )(_p1_body)
    scr_u, scr_i, scr_d = p1(user1, item1, age1,
                             user_table.T, item_table.T, item_dyn_table.T)

    tail_u = user_table[U_TAIL:]
    tail_i = item_table[I_TAIL:]
    tail_d = item_dyn_table[U_TAIL:]

    per_w = B // NW
    p2 = functools.partial(
        pl.kernel,
        out_type=jax.ShapeDtypeStruct((B,), jnp.float32),
        mesh=mesh,
        compiler_params=params,
        scratch_types=[
            pltpu.VMEM((per_w,), jnp.int32),
            pltpu.VMEM((per_w,), jnp.int32),
            pltpu.VMEM((per_w,), jnp.int32),
            pltpu.VMEM((N_USERS - U_TAIL, EMB), jnp.float32),
            pltpu.VMEM((N_ITEMS - I_TAIL, EMB), jnp.float32),
            pltpu.VMEM((N_DYN - U_TAIL, EMB), jnp.float32),
            pltpu.VMEM((per_w // 2, 128), jnp.float32),
            pltpu.VMEM((per_w // 2, 128), jnp.float32),
            pltpu.VMEM((per_w // 2, 128), jnp.float32),
            pltpu.VMEM((per_w,), jnp.float32),
        ],
    )(_p2_body)
    return p2(user1, item1, age1, scr_u, scr_i, scr_d,
              tail_u, tail_i, tail_d)
